# baseline (speedup 1.0000x reference)
import functools

import jax
import jax.numpy as jnp
from jax import lax
from jax.experimental import pallas as pl
from jax.experimental.pallas import tpu as pltpu
from jax.experimental.pallas import tpu_sc as plsc

VOCAB = 19240
EMB = 10
BATCH = 4096
DP = 128
VB = 512
NMAIN = VOCAB // VB
TAIL = VOCAB - NMAIN * VB
NSPLIT = 8
VSUB = VB // NSPLIT


def _prep_body(tabt_ref, w1_ref, out_hbm, buf, sems):
    i = pl.program_id(0)
    n = pl.num_programs(0)
    slot = lax.rem(i, 2)

    def _copy(step, s, size):
        return pltpu.make_async_copy(
            buf.at[s, pl.ds(0, size)],
            out_hbm.at[pl.ds(step * VB, size)],
            sems.at[s],
        )

    @pl.when(i >= 2)
    def _reclaim():
        _copy(i - 2, slot, VB).wait()

    def _compute(ts, size):
        h = lax.dot_general(w1_ref[...], ts, (((1,), (0,)), ((), ())),
                            preferred_element_type=jnp.float32)
        ht = jnp.maximum(h, 0.0).T
        return jnp.concatenate([ht, jnp.zeros((size, DP - EMB), jnp.float32)],
                               axis=1)

    @pl.when(i < NMAIN)
    def _main():
        buf[slot] = _compute(tabt_ref[:, pl.ds(i * VB, VB)], VB)
        _copy(i, slot, VB).start()

    @pl.when(i == NMAIN)
    def _tail():
        buf[slot, pl.ds(0, TAIL)] = _compute(
            tabt_ref[:, pl.ds(NMAIN * VB, TAIL)], TAIL)
        _copy(i, slot, TAIL).start()

    @pl.when(i == n - 1)
    def _drain():
        _copy(i - 1, 1 - slot, VB).wait()
        _copy(i, slot, TAIL).wait()


def _prep(tabt, w1):
    return pl.pallas_call(
        _prep_body,
        grid=(NMAIN + 1,),
        in_specs=[
            pl.BlockSpec((EMB, VOCAB), lambda i: (0, 0)),
            pl.BlockSpec((EMB, EMB), lambda i: (0, 0)),
        ],
        out_specs=pl.BlockSpec(memory_space=pltpu.MemorySpace.HBM),
        out_shape=jax.ShapeDtypeStruct((VOCAB, DP), jnp.float32),
        scratch_shapes=[
            pltpu.VMEM((2, VB, DP), jnp.float32),
            pltpu.SemaphoreType.DMA((2,)),
        ],
    )(tabt, w1)


def _make_sc_gather():
    info = plsc.get_sparse_core_info()
    nc, ns = info.num_cores, info.num_subcores
    nw = nc * ns
    bpw = BATCH // nw
    mesh = plsc.VectorSubcoreMesh(core_axis_name="c", subcore_axis_name="s")

    @functools.partial(
        pl.kernel,
        mesh=mesh,
        out_type=jax.ShapeDtypeStruct((BATCH, DP), jnp.float32),
        scratch_types=[
            pltpu.VMEM((bpw,), jnp.int32),
            pltpu.VMEM((bpw, DP), jnp.float32),
            pltpu.SemaphoreType.DMA,
        ],
    )
    def sc_gather(table_hbm, idx_hbm, out_hbm, idx_v, rows_v, sem):
        wid = lax.axis_index("s") * nc + lax.axis_index("c")
        base = wid * bpw
        pltpu.sync_copy(idx_hbm.at[pl.ds(base, bpw)], idx_v)
        pltpu.async_copy(table_hbm.at[idx_v], rows_v, sem).wait()
        pltpu.sync_copy(rows_v, out_hbm.at[pl.ds(base, bpw)])

    return sc_gather


def _tc_body(hidp_ref, w2t_ref, out_hbm, buf, hid, sems):
    i = pl.program_id(0)
    n = pl.num_programs(0)
    slot = lax.rem(i, 2)

    @pl.when(i == 0)
    def _prologue():
        hid[...] = hidp_ref[...].T[:EMB]

    def _copies(step, s):
        return [
            pltpu.make_async_copy(
                buf.at[s, pl.ds(j * VSUB, VSUB)],
                out_hbm.at[pl.ds(step * VB + j * VSUB, VSUB)],
                sems.at[s, j],
            )
            for j in range(NSPLIT)
        ]

    def _tail_copy(s):
        return pltpu.make_async_copy(
            buf.at[s, pl.ds(0, TAIL)],
            out_hbm.at[pl.ds(NMAIN * VB, TAIL)],
            sems.at[s, NSPLIT],
        )

    @pl.when(i >= 2)
    def _reclaim():
        for c in _copies(i - 2, slot):
            c.wait()

    @pl.when(i < NMAIN)
    def _main():
        w2t_slice = w2t_ref[:, pl.ds(i * VB, VB)]
        buf[slot] = lax.dot_general(w2t_slice, hid[...],
                                    (((0,), (0,)), ((), ())),
                                    preferred_element_type=jnp.float32)
        for c in _copies(i, slot):
            c.start()

    @pl.when(i == NMAIN)
    def _tail():
        w2t_slice = w2t_ref[:, pl.ds(NMAIN * VB, TAIL)]
        buf[slot, pl.ds(0, TAIL)] = lax.dot_general(
            w2t_slice, hid[...], (((0,), (0,)), ((), ())),
            preferred_element_type=jnp.float32)
        _tail_copy(slot).start()

    @pl.when(i == n - 1)
    def _drain():
        for c in _copies(i - 1, 1 - slot):
            c.wait()
        _tail_copy(slot).wait()


def _tc_mlp(hidp, w2t):
    return pl.pallas_call(
        _tc_body,
        grid=(NMAIN + 1,),
        in_specs=[
            pl.BlockSpec((BATCH, DP), lambda i: (0, 0)),
            pl.BlockSpec((EMB, VOCAB), lambda i: (0, 0)),
        ],
        out_specs=pl.BlockSpec(memory_space=pltpu.MemorySpace.HBM),
        out_shape=jax.ShapeDtypeStruct((VOCAB, BATCH), jnp.float32),
        scratch_shapes=[
            pltpu.VMEM((2, VB, BATCH), jnp.float32),
            pltpu.VMEM((EMB, BATCH), jnp.float32),
            pltpu.SemaphoreType.DMA((2, NSPLIT + 1)),
        ],
    )(hidp, w2t)


@jax.jit
def kernel(X, emb_table, W1, W2):
    X = X.astype(jnp.int32)
    h_all = _prep(emb_table.T, W1)
    hidp = _make_sc_gather()(h_all, X)
    return _tc_mlp(hidp, W2.T).T


# speedup vs baseline: 1.0808x; 1.0808x over previous
import functools

import jax
import jax.numpy as jnp
from jax import lax
from jax.experimental import pallas as pl
from jax.experimental.pallas import tpu as pltpu
from jax.experimental.pallas import tpu_sc as plsc

VOCAB = 19240
EMB = 10
BATCH = 4096
DP = 128
VB = 512
NMAIN = VOCAB // VB
TAIL = VOCAB - NMAIN * VB
NSPLIT = 4
VSUB = VB // NSPLIT
PV = 1024
NPREP = VOCAB // PV
PTAIL = VOCAB - NPREP * PV
PBUF = 4


def _prep_body(tabt_ref, w1_ref, out_hbm, buf, sems):
    i = pl.program_id(0)
    n = pl.num_programs(0)
    slot = lax.rem(i, PBUF)

    def _copy(step, s, size):
        return pltpu.make_async_copy(
            buf.at[s, pl.ds(0, size)],
            out_hbm.at[pl.ds(step * PV, size)],
            sems.at[s],
        )

    @pl.when(i >= PBUF)
    def _reclaim():
        _copy(i - PBUF, slot, PV).wait()

    def _compute(ts, size):
        h = lax.dot_general(w1_ref[...], ts, (((1,), (0,)), ((), ())),
                            preferred_element_type=jnp.float32)
        ht = jnp.maximum(h, 0.0).T
        return jnp.concatenate([ht, jnp.zeros((size, DP - EMB), jnp.float32)],
                               axis=1)

    @pl.when(i < NPREP)
    def _main():
        buf[slot] = _compute(tabt_ref[:, pl.ds(i * PV, PV)], PV)
        _copy(i, slot, PV).start()

    @pl.when(i == NPREP)
    def _tail():
        buf[slot, pl.ds(0, PTAIL)] = _compute(
            tabt_ref[:, pl.ds(NPREP * PV, PTAIL)], PTAIL)
        _copy(i, slot, PTAIL).start()

    @pl.when(i == n - 1)
    def _drain():
        for k in range(1, PBUF):
            _copy(i - k, lax.rem(i - k + PBUF, PBUF), PV).wait()
        _copy(i, slot, PTAIL).wait()


def _prep(tabt, w1):
    return pl.pallas_call(
        _prep_body,
        grid=(NPREP + 1,),
        in_specs=[
            pl.BlockSpec((EMB, VOCAB), lambda i: (0, 0)),
            pl.BlockSpec((EMB, EMB), lambda i: (0, 0)),
        ],
        out_specs=pl.BlockSpec(memory_space=pltpu.MemorySpace.HBM),
        out_shape=jax.ShapeDtypeStruct((VOCAB, DP), jnp.float32),
        scratch_shapes=[
            pltpu.VMEM((PBUF, PV, DP), jnp.float32),
            pltpu.SemaphoreType.DMA((PBUF,)),
        ],
    )(tabt, w1)


def _make_sc_gather():
    info = plsc.get_sparse_core_info()
    nc, ns = info.num_cores, info.num_subcores
    nw = nc * ns
    bpw = BATCH // nw
    mesh = plsc.VectorSubcoreMesh(core_axis_name="c", subcore_axis_name="s")

    @functools.partial(
        pl.kernel,
        mesh=mesh,
        out_type=jax.ShapeDtypeStruct((BATCH, DP), jnp.float32),
        scratch_types=[
            pltpu.VMEM((bpw,), jnp.int32),
            pltpu.VMEM((bpw, DP), jnp.float32),
            pltpu.SemaphoreType.DMA,
        ],
    )
    def sc_gather(table_hbm, idx_hbm, out_hbm, idx_v, rows_v, sem):
        wid = lax.axis_index("s") * nc + lax.axis_index("c")
        base = wid * bpw
        pltpu.sync_copy(idx_hbm.at[pl.ds(base, bpw)], idx_v)
        pltpu.async_copy(table_hbm.at[idx_v], rows_v, sem).wait()
        pltpu.sync_copy(rows_v, out_hbm.at[pl.ds(base, bpw)])

    return sc_gather


def _tc_body(hidp_ref, w2t_ref, out_hbm, buf, hid, sems):
    i = pl.program_id(0)
    n = pl.num_programs(0)
    slot = lax.rem(i, 2)

    @pl.when(i == 0)
    def _prologue():
        hid[...] = hidp_ref[...].T[:EMB]

    def _copies(step, s):
        return [
            pltpu.make_async_copy(
                buf.at[s, pl.ds(j * VSUB, VSUB)],
                out_hbm.at[pl.ds(step * VB + j * VSUB, VSUB)],
                sems.at[s, j],
            )
            for j in range(NSPLIT)
        ]

    def _tail_copy(s):
        return pltpu.make_async_copy(
            buf.at[s, pl.ds(0, TAIL)],
            out_hbm.at[pl.ds(NMAIN * VB, TAIL)],
            sems.at[s, NSPLIT],
        )

    @pl.when(i >= 2)
    def _reclaim():
        for c in _copies(i - 2, slot):
            c.wait()

    @pl.when(i < NMAIN)
    def _main():
        w2t_slice = w2t_ref[:, pl.ds(i * VB, VB)]
        buf[slot] = lax.dot_general(w2t_slice, hid[...],
                                    (((0,), (0,)), ((), ())),
                                    preferred_element_type=jnp.float32)
        for c in _copies(i, slot):
            c.start()

    @pl.when(i == NMAIN)
    def _tail():
        w2t_slice = w2t_ref[:, pl.ds(NMAIN * VB, TAIL)]
        buf[slot, pl.ds(0, TAIL)] = lax.dot_general(
            w2t_slice, hid[...], (((0,), (0,)), ((), ())),
            preferred_element_type=jnp.float32)
        _tail_copy(slot).start()

    @pl.when(i == n - 1)
    def _drain():
        for c in _copies(i - 1, 1 - slot):
            c.wait()
        _tail_copy(slot).wait()


def _tc_mlp(hidp, w2t):
    return pl.pallas_call(
        _tc_body,
        grid=(NMAIN + 1,),
        in_specs=[
            pl.BlockSpec((BATCH, DP), lambda i: (0, 0)),
            pl.BlockSpec((EMB, VOCAB), lambda i: (0, 0)),
        ],
        out_specs=pl.BlockSpec(memory_space=pltpu.MemorySpace.HBM),
        out_shape=jax.ShapeDtypeStruct((VOCAB, BATCH), jnp.float32),
        scratch_shapes=[
            pltpu.VMEM((2, VB, BATCH), jnp.float32),
            pltpu.VMEM((EMB, BATCH), jnp.float32),
            pltpu.SemaphoreType.DMA((2, NSPLIT + 1)),
        ],
    )(hidp, w2t)


@jax.jit
def kernel(X, emb_table, W1, W2):
    X = X.astype(jnp.int32)
    h_all = _prep(emb_table.T, W1)
    hidp = _make_sc_gather()(h_all, X)
    return _tc_mlp(hidp, W2.T).T


# speedup vs baseline: 1.0999x; 1.0176x over previous
import functools

import jax
import jax.numpy as jnp
from jax import lax
from jax.experimental import pallas as pl
from jax.experimental.pallas import tpu as pltpu
from jax.experimental.pallas import tpu_sc as plsc

VOCAB = 19240
EMB = 10
BATCH = 4096
DP = 128
VB = 512
NMAIN = VOCAB // VB
TAIL = VOCAB - NMAIN * VB
NSPLIT = 4
VSUB = VB // NSPLIT
PV = 2048
NPREP = VOCAB // PV
PTAIL = VOCAB - NPREP * PV
PBUF = 4
NBUF = 3


def _prep_body(tabt_ref, w1_ref, out_hbm, buf, sems):
    i = pl.program_id(0)
    n = pl.num_programs(0)
    slot = lax.rem(i, PBUF)

    def _copy(step, s, size):
        return pltpu.make_async_copy(
            buf.at[s, pl.ds(0, size)],
            out_hbm.at[pl.ds(step * PV, size)],
            sems.at[s],
        )

    @pl.when(i >= PBUF)
    def _reclaim():
        _copy(i - PBUF, slot, PV).wait()

    def _compute(ts):
        h = lax.dot_general(w1_ref[...], ts, (((1,), (0,)), ((), ())),
                            preferred_element_type=jnp.float32)
        return jnp.maximum(h, 0.0).T

    @pl.when(i < NPREP)
    def _main():
        buf[slot, :, :EMB] = _compute(tabt_ref[:, pl.ds(i * PV, PV)])
        _copy(i, slot, PV).start()

    @pl.when(i == NPREP)
    def _tail():
        buf[slot, pl.ds(0, PTAIL), :EMB] = _compute(
            tabt_ref[:, pl.ds(NPREP * PV, PTAIL)])
        _copy(i, slot, PTAIL).start()

    @pl.when(i == n - 1)
    def _drain():
        for k in range(1, PBUF):
            _copy(i - k, lax.rem(i - k + PBUF, PBUF), PV).wait()
        _copy(i, slot, PTAIL).wait()


def _prep(tabt, w1):
    return pl.pallas_call(
        _prep_body,
        grid=(NPREP + 1,),
        in_specs=[
            pl.BlockSpec((EMB, VOCAB), lambda i: (0, 0)),
            pl.BlockSpec((EMB, EMB), lambda i: (0, 0)),
        ],
        out_specs=pl.BlockSpec(memory_space=pltpu.MemorySpace.HBM),
        out_shape=jax.ShapeDtypeStruct((VOCAB, DP), jnp.float32),
        scratch_shapes=[
            pltpu.VMEM((PBUF, PV, DP), jnp.float32),
            pltpu.SemaphoreType.DMA((PBUF,)),
        ],
    )(tabt, w1)


def _make_sc_gather():
    info = plsc.get_sparse_core_info()
    nc, ns = info.num_cores, info.num_subcores
    nw = nc * ns
    bpw = BATCH // nw
    mesh = plsc.VectorSubcoreMesh(core_axis_name="c", subcore_axis_name="s")

    @functools.partial(
        pl.kernel,
        mesh=mesh,
        out_type=jax.ShapeDtypeStruct((BATCH, DP), jnp.float32),
        scratch_types=[
            pltpu.VMEM((bpw,), jnp.int32),
            pltpu.VMEM((bpw, DP), jnp.float32),
            pltpu.SemaphoreType.DMA,
        ],
    )
    def sc_gather(table_hbm, idx_hbm, out_hbm, idx_v, rows_v, sem):
        wid = lax.axis_index("s") * nc + lax.axis_index("c")
        base = wid * bpw
        pltpu.sync_copy(idx_hbm.at[pl.ds(base, bpw)], idx_v)
        pltpu.async_copy(table_hbm.at[idx_v], rows_v, sem).wait()
        pltpu.sync_copy(rows_v, out_hbm.at[pl.ds(base, bpw)])

    return sc_gather


def _tc_body(hidp_ref, w2t_ref, out_hbm, buf, hid, sems):
    i = pl.program_id(0)
    n = pl.num_programs(0)
    slot = lax.rem(i, NBUF)

    @pl.when(i == 0)
    def _prologue():
        hid[...] = hidp_ref[...].T[:EMB]

    def _copies(step, s):
        return [
            pltpu.make_async_copy(
                buf.at[s, pl.ds(j * VSUB, VSUB)],
                out_hbm.at[pl.ds(step * VB + j * VSUB, VSUB)],
                sems.at[s, j],
            )
            for j in range(NSPLIT)
        ]

    def _tail_copy(s):
        return pltpu.make_async_copy(
            buf.at[s, pl.ds(0, TAIL)],
            out_hbm.at[pl.ds(NMAIN * VB, TAIL)],
            sems.at[s, NSPLIT],
        )

    @pl.when(i >= NBUF)
    def _reclaim():
        for c in _copies(i - NBUF, slot):
            c.wait()

    @pl.when(i < NMAIN)
    def _main():
        w2t_slice = w2t_ref[:, pl.ds(i * VB, VB)]
        buf[slot] = lax.dot_general(w2t_slice, hid[...],
                                    (((0,), (0,)), ((), ())),
                                    preferred_element_type=jnp.float32)
        for c in _copies(i, slot):
            c.start()

    @pl.when(i == NMAIN)
    def _tail():
        w2t_slice = w2t_ref[:, pl.ds(NMAIN * VB, TAIL)]
        buf[slot, pl.ds(0, TAIL)] = lax.dot_general(
            w2t_slice, hid[...], (((0,), (0,)), ((), ())),
            preferred_element_type=jnp.float32)
        _tail_copy(slot).start()

    @pl.when(i == n - 1)
    def _drain():
        for k in range(1, NBUF):
            for c in _copies(i - k, lax.rem(i - k + NBUF, NBUF)):
                c.wait()
        _tail_copy(slot).wait()


def _tc_mlp(hidp, w2t):
    return pl.pallas_call(
        _tc_body,
        grid=(NMAIN + 1,),
        in_specs=[
            pl.BlockSpec((BATCH, DP), lambda i: (0, 0)),
            pl.BlockSpec((EMB, VOCAB), lambda i: (0, 0)),
        ],
        out_specs=pl.BlockSpec(memory_space=pltpu.MemorySpace.HBM),
        out_shape=jax.ShapeDtypeStruct((VOCAB, BATCH), jnp.float32),
        scratch_shapes=[
            pltpu.VMEM((NBUF, VB, BATCH), jnp.float32),
            pltpu.VMEM((EMB, BATCH), jnp.float32),
            pltpu.SemaphoreType.DMA((NBUF, NSPLIT + 1)),
        ],
    )(hidp, w2t)


@jax.jit
def kernel(X, emb_table, W1, W2):
    X = X.astype(jnp.int32)
    h_all = _prep(emb_table.T, W1)
    hidp = _make_sc_gather()(h_all, X)
    return _tc_mlp(hidp, W2.T).T
